# baseline (speedup 1.0000x reference)
import jax
import jax.numpy as jnp
from jax import lax
from jax.experimental import pallas as pl
from jax.experimental.pallas import tpu as pltpu
from jax.experimental.pallas import tpu_sc as plsc

N, C, H, W = 4, 3, 544, 960
HW = H * W
NPTS = N * HW
TOTAL = N * C * HW
NC, NS = 2, 16
NW = NC * NS
PTS_W = NPTS // NW
WPS = NS // 2
BLK = 768
NBLK = PTS_W // BLK
GCH = 128
NG = BLK // GCH
PBLK = 1920
PNBLK = PTS_W // PBLK


def _body(x_hbm, g_hbm, out_hbm, xp,
          s0v, s1v, s2v, xpb,
          gbuf, i0v, i1v, w00v, w01v, w10v, w11v, val0, val1, outv, sem):
    cid = lax.axis_index("c")
    sid = lax.axis_index("s")
    n = 2 * cid + sid // WPS
    chunk = sid % WPS
    pix_base = n * HW + chunk * PTS_W

    strips = (s0v, s1v, s2v)

    def p1blk(b, _):
        q0 = chunk * PTS_W + b * PBLK
        for c in range(C):
            so = (n * C + c) * HW + q0
            pltpu.sync_copy(x_hbm.at[pl.ds(so, PBLK)],
                            strips[c].at[pl.ds(0, PBLK)])
            tail = jnp.minimum(so + PBLK, TOTAL - 8)
            pltpu.sync_copy(x_hbm.at[pl.ds(tail, 8)],
                            strips[c].at[pl.ds(PBLK, 8)])

        def ilv(t, _):
            rows = lax.iota(jnp.int32, 16) + t * 16
            for c in range(C):
                v0 = strips[c][pl.ds(t * 16, 16)]
                v1 = strips[c][pl.ds(t * 16 + 1, 16)]
                plsc.store_scatter(xpb, [rows, jnp.full((16,), c, jnp.int32)], v0)
                plsc.store_scatter(
                    xpb, [rows, jnp.full((16,), c + C, jnp.int32)], v1)
            return _

        lax.fori_loop(0, PBLK // 16, ilv, None)
        pltpu.sync_copy(xpb, xp.at[pl.ds(pix_base + b * PBLK, PBLK)])
        return _

    lax.fori_loop(0, PNBLK, p1blk, None)
    plsc.subcore_barrier()

    xp2 = xp

    def blk(b, _):
        off = pix_base + b * BLK
        pltpu.sync_copy(g_hbm.at[pl.ds(2 * off, 2 * BLK)], gbuf)

        def idxw(j, _):
            sl = pl.ds(j * 16, 16)
            lane2 = lax.iota(jnp.int32, 16) * 2 + j * 32
            gx = plsc.load_gather(gbuf, [lane2])
            gy = plsc.load_gather(gbuf, [lane2 + 1])
            gx = jnp.minimum(jnp.maximum(gx, -1.0), 1.0)
            gy = jnp.minimum(jnp.maximum(gy, -1.0), 1.0)
            ix = (gx + 1.0) * 0.5 * (W - 1)
            iy = (gy + 1.0) * 0.5 * (H - 1)
            x0 = ix.astype(jnp.int32)
            y0 = iy.astype(jnp.int32)
            wx1 = ix - x0.astype(jnp.float32)
            wy1 = iy - y0.astype(jnp.float32)
            wx0 = 1.0 - wx1
            wy0 = 1.0 - wy1
            row = y0 * W + x0 + n * HW
            i0v[sl] = row
            i1v[sl] = row + jnp.where(y0 < H - 1, W, 0)
            w00v[sl] = wy0 * wx0
            w01v[sl] = wy0 * wx1
            w10v[sl] = wy1 * wx0
            w11v[sl] = wy1 * wx1
            return _

        lax.fori_loop(0, BLK // 16, idxw, None)

        copies = []
        for k in range(NG):
            ks = pl.ds(k * GCH, GCH)
            copies.append(pltpu.async_copy(xp2.at[i0v.at[ks]], val0.at[ks], sem))
            copies.append(pltpu.async_copy(xp2.at[i1v.at[ks]], val1.at[ks], sem))
        for cp in copies:
            cp.wait()

        def comb(j, _):
            sl = pl.ds(j * 16, 16)
            rows = lax.iota(jnp.int32, 16) + j * 16
            w00 = w00v[sl]
            w01 = w01v[sl]
            w10 = w10v[sl]
            w11 = w11v[sl]
            for comp in range(C):
                c0 = jnp.full((16,), comp, jnp.int32)
                c1 = jnp.full((16,), comp + C, jnp.int32)
                v00 = plsc.load_gather(val0, [rows, c0])
                v01 = plsc.load_gather(val0, [rows, c1])
                v10 = plsc.load_gather(val1, [rows, c0])
                v11 = plsc.load_gather(val1, [rows, c1])
                outv[pl.ds(comp * BLK + j * 16, 16)] = (
                    (v00 * w00 + v01 * w01) + (v10 * w10 + v11 * w11))
            return _

        lax.fori_loop(0, BLK // 16, comb, None)

        for comp in range(C):
            o = (n * C + comp) * HW + chunk * PTS_W + b * BLK
            pltpu.sync_copy(outv.at[pl.ds(comp * BLK, BLK)],
                            out_hbm.at[pl.ds(o, BLK)])
        return _

    lax.fori_loop(0, NBLK, blk, None)


_sc_call = pl.kernel(
    _body,
    out_type=(jax.ShapeDtypeStruct((TOTAL,), jnp.float32),
              jax.ShapeDtypeStruct((NPTS, 8), jnp.float32)),
    mesh=plsc.VectorSubcoreMesh(
        core_axis_name="c", subcore_axis_name="s",
        num_cores=NC, num_subcores=NS),
    scratch_types=[
        pltpu.VMEM((PBLK + 8,), jnp.float32),
        pltpu.VMEM((PBLK + 8,), jnp.float32),
        pltpu.VMEM((PBLK + 8,), jnp.float32),
        pltpu.VMEM((PBLK, 8), jnp.float32),
        pltpu.VMEM((2 * BLK,), jnp.float32),
        pltpu.VMEM((BLK,), jnp.int32),
        pltpu.VMEM((BLK,), jnp.int32),
        pltpu.VMEM((BLK,), jnp.float32),
        pltpu.VMEM((BLK,), jnp.float32),
        pltpu.VMEM((BLK,), jnp.float32),
        pltpu.VMEM((BLK,), jnp.float32),
        pltpu.VMEM((BLK, 8), jnp.float32),
        pltpu.VMEM((BLK, 8), jnp.float32),
        pltpu.VMEM((C * BLK,), jnp.float32),
        pltpu.SemaphoreType.DMA,
    ],
    compiler_params=pltpu.CompilerParams(
        needs_layout_passes=False, use_tc_tiling_on_sc=False),
)


def kernel(x, grid):
    out_flat, _ = _sc_call(x.reshape(TOTAL), grid.reshape(NPTS * 2))
    return out_flat.reshape(N, C, H, W)


# speedup vs baseline: 1.0051x; 1.0051x over previous
import jax
import jax.numpy as jnp
from jax import lax
from jax.experimental import pallas as pl
from jax.experimental.pallas import tpu as pltpu
from jax.experimental.pallas import tpu_sc as plsc

N, C, H, W = 4, 3, 544, 960
HW = H * W
NPTS = N * HW
TOTAL = N * C * HW
NC, NS = 2, 16
NW = NC * NS
PTS_W = NPTS // NW
WPS = NS // 2
BLK = 768
NBLK = PTS_W // BLK
GCH = 128
NG = BLK // GCH
PBLK = 1920
PNBLK = PTS_W // PBLK


def _body(x_hbm, g_hbm, out_hbm,
          xp, s0v, s1v, s2v, xpb,
          gbuf, i0v, i1v, w00v, w01v, w10v, w11v, val0, val1, outv, sem):
    cid = lax.axis_index("c")
    sid = lax.axis_index("s")
    n = 2 * cid + sid // WPS
    chunk = sid % WPS
    pix_base = n * HW + chunk * PTS_W

    strips = (s0v, s1v, s2v)

    def p1blk(b, _):
        q0 = chunk * PTS_W + b * PBLK
        for c in range(C):
            so = (n * C + c) * HW + q0
            pltpu.sync_copy(x_hbm.at[pl.ds(so, PBLK)],
                            strips[c].at[pl.ds(0, PBLK)])
            tail = jnp.minimum(so + PBLK, TOTAL - 8)
            pltpu.sync_copy(x_hbm.at[pl.ds(tail, 8)],
                            strips[c].at[pl.ds(PBLK, 8)])

        def ilv(t, _):
            rows = lax.iota(jnp.int32, 16) + t * 16
            for c in range(C):
                v0 = strips[c][pl.ds(t * 16, 16)]
                v1 = strips[c][pl.ds(t * 16 + 1, 16)]
                plsc.store_scatter(xpb, [rows, jnp.full((16,), c, jnp.int32)], v0)
                plsc.store_scatter(
                    xpb, [rows, jnp.full((16,), c + C, jnp.int32)], v1)
            return _

        lax.fori_loop(0, PBLK // 16, ilv, None)
        pltpu.sync_copy(xpb, xp.at[pl.ds(pix_base + b * PBLK, PBLK)])
        return _

    lax.fori_loop(0, PNBLK, p1blk, None)
    plsc.subcore_barrier()

    xp2 = xp

    def blk(b, _):
        off = pix_base + b * BLK
        pltpu.sync_copy(g_hbm.at[pl.ds(2 * off, 2 * BLK)], gbuf)

        def idxw(j, _):
            sl = pl.ds(j * 16, 16)
            lane2 = lax.iota(jnp.int32, 16) * 2 + j * 32
            gx = plsc.load_gather(gbuf, [lane2])
            gy = plsc.load_gather(gbuf, [lane2 + 1])
            gx = jnp.minimum(jnp.maximum(gx, -1.0), 1.0)
            gy = jnp.minimum(jnp.maximum(gy, -1.0), 1.0)
            ix = (gx + 1.0) * 0.5 * (W - 1)
            iy = (gy + 1.0) * 0.5 * (H - 1)
            x0 = ix.astype(jnp.int32)
            y0 = iy.astype(jnp.int32)
            wx1 = ix - x0.astype(jnp.float32)
            wy1 = iy - y0.astype(jnp.float32)
            wx0 = 1.0 - wx1
            wy0 = 1.0 - wy1
            row = y0 * W + x0 + n * HW
            i0v[sl] = row
            i1v[sl] = row + jnp.where(y0 < H - 1, W, 0)
            w00v[sl] = wy0 * wx0
            w01v[sl] = wy0 * wx1
            w10v[sl] = wy1 * wx0
            w11v[sl] = wy1 * wx1
            return _

        lax.fori_loop(0, BLK // 16, idxw, None)

        copies = []
        for k in range(NG):
            ks = pl.ds(k * GCH, GCH)
            copies.append(pltpu.async_copy(xp2.at[i0v.at[ks]], val0.at[ks], sem))
            copies.append(pltpu.async_copy(xp2.at[i1v.at[ks]], val1.at[ks], sem))
        for cp in copies:
            cp.wait()

        def comb(j, _):
            sl = pl.ds(j * 16, 16)
            rows = lax.iota(jnp.int32, 16) + j * 16
            w00 = w00v[sl]
            w01 = w01v[sl]
            w10 = w10v[sl]
            w11 = w11v[sl]
            for comp in range(C):
                c0 = jnp.full((16,), comp, jnp.int32)
                c1 = jnp.full((16,), comp + C, jnp.int32)
                v00 = plsc.load_gather(val0, [rows, c0])
                v01 = plsc.load_gather(val0, [rows, c1])
                v10 = plsc.load_gather(val1, [rows, c0])
                v11 = plsc.load_gather(val1, [rows, c1])
                outv[pl.ds(comp * BLK + j * 16, 16)] = (
                    (v00 * w00 + v01 * w01) + (v10 * w10 + v11 * w11))
            return _

        lax.fori_loop(0, BLK // 16, comb, None)

        for comp in range(C):
            o = (n * C + comp) * HW + chunk * PTS_W + b * BLK
            pltpu.sync_copy(outv.at[pl.ds(comp * BLK, BLK)],
                            out_hbm.at[pl.ds(o, BLK)])
        return _

    lax.fori_loop(0, NBLK, blk, None)


_sc_call = pl.kernel(
    _body,
    out_type=jax.ShapeDtypeStruct((TOTAL,), jnp.float32),
    mesh=plsc.VectorSubcoreMesh(
        core_axis_name="c", subcore_axis_name="s",
        num_cores=NC, num_subcores=NS),
    scratch_types=[
        pltpu.HBM((NPTS, 8), jnp.float32),
        pltpu.VMEM((PBLK + 8,), jnp.float32),
        pltpu.VMEM((PBLK + 8,), jnp.float32),
        pltpu.VMEM((PBLK + 8,), jnp.float32),
        pltpu.VMEM((PBLK, 8), jnp.float32),
        pltpu.VMEM((2 * BLK,), jnp.float32),
        pltpu.VMEM((BLK,), jnp.int32),
        pltpu.VMEM((BLK,), jnp.int32),
        pltpu.VMEM((BLK,), jnp.float32),
        pltpu.VMEM((BLK,), jnp.float32),
        pltpu.VMEM((BLK,), jnp.float32),
        pltpu.VMEM((BLK,), jnp.float32),
        pltpu.VMEM((BLK, 8), jnp.float32),
        pltpu.VMEM((BLK, 8), jnp.float32),
        pltpu.VMEM((C * BLK,), jnp.float32),
        pltpu.SemaphoreType.DMA,
    ],
    compiler_params=pltpu.CompilerParams(
        needs_layout_passes=False, use_tc_tiling_on_sc=False),
)


def kernel(x, grid):
    out_flat = _sc_call(x.reshape(TOTAL), grid.reshape(NPTS * 2))
    return out_flat.reshape(N, C, H, W)
